# baseline (speedup 1.0000x reference)
import functools

import jax
import jax.numpy as jnp
from jax import lax
from jax.experimental import pallas as pl
from jax.experimental.pallas import tpu as pltpu
from jax.experimental.pallas import tpu_sc as plsc

HIDDEN = 128
EPS = 1e-12

NC, NS = 2, 16
NW = NC * NS
P_TOTAL = 4096 * 100
P_PER_W = P_TOTAL // NW
G = 128
NG = P_PER_W // G
REP = 32
NB = 2
NR = NG // NB
PW = 2 * HIDDEN


def _ln_body(t_ref, g_ref, b_ref, o_ref):
    t = t_ref[...]
    mean = jnp.mean(t, axis=-1, keepdims=True)
    c = t - mean
    var = jnp.mean(c * c, axis=-1, keepdims=True)
    n = c * lax.rsqrt(var + EPS) * g_ref[...] + b_ref[...]
    left = jnp.broadcast_to(n[:, None, :], (4, 4, HIDDEN)).reshape(16, HIDDEN)
    right = jnp.broadcast_to(n[None, :, :], (4, 4, HIDDEN)).reshape(16, HIDDEN)
    pair = jnp.stack([left, right], axis=1)
    o_ref[...] = jnp.broadcast_to(pair[None], (REP, 16, 2, HIDDEN))


def _pair_table(table, gamma, beta):
    out = pl.pallas_call(
        _ln_body,
        out_shape=jax.ShapeDtypeStruct((REP, 16, 2, HIDDEN), jnp.float32),
    )(table, gamma.reshape(1, HIDDEN), beta.reshape(1, HIDDEN))
    return out.reshape(16 * REP, 2, HIDDEN)


def _gather_body(idx_hbm, table_hbm, out_hbm, idx_v, rows_v, spm_table,
                 gsem, ssem, isem):
    sid = lax.axis_index("s")
    wid = sid * NC + lax.axis_index("c")
    base = wid * P_PER_W
    rows_per_sub = 16 * REP // NS
    pltpu.sync_copy(
        table_hbm.at[pl.ds(sid * rows_per_sub, rows_per_sub)],
        spm_table.at[pl.ds(sid * rows_per_sub, rows_per_sub)])

    pltpu.sync_copy(idx_hbm.at[wid, pl.ds(0, 8)], idx_v.at[pl.ds(0, 8)])
    pltpu.async_copy(
        idx_hbm.at[wid, pl.ds(8, NG - 8)], idx_v.at[pl.ds(8, NG - 8)], isem)
    plsc.subcore_barrier()

    lanes = lax.iota(jnp.int32, 16)

    def fix_row(t):
        for c in range(8):
            sl = pl.ds(c * 16, 16)
            idx_v[t, sl] = idx_v[t, sl] + (lanes * 16 + (c % 2) * 256)

    def gstart(b, j):
        pltpu.async_copy(spm_table.at[idx_v.at[j]], rows_v.at[b], gsem.at[b])

    def gwait(b, j):
        pltpu.make_async_copy(
            spm_table.at[idx_v.at[j]], rows_v.at[b], gsem.at[b]).wait()

    def sstart(b, j):
        pltpu.async_copy(
            rows_v.at[b], out_hbm.at[pl.ds(base + j * G, G)], ssem.at[b])

    def swait(b, j):
        pltpu.make_async_copy(
            rows_v.at[b], out_hbm.at[pl.ds(base + j * G, G)], ssem.at[b]).wait()

    for b in range(NB):
        fix_row(b)
        gstart(b, b)

    pltpu.make_async_copy(
        idx_hbm.at[wid, pl.ds(8, NG - 8)], idx_v.at[pl.ds(8, NG - 8)],
        isem).wait()

    def round_(r, _):
        j0 = r * NB
        for b in range(NB):
            j = j0 + b
            fix_row(j + NB)
            gwait(b, j)
            sstart(b, j)
            swait(b, j)
            gstart(b, j + NB)
        return 0

    lax.fori_loop(0, NR - 1, round_, 0)

    j0 = (NR - 1) * NB
    for b in range(NB):
        gwait(b, j0 + b)
        sstart(b, j0 + b)
        swait(b, j0 + b)


_gather = functools.partial(
    pl.kernel,
    out_type=jax.ShapeDtypeStruct((P_TOTAL, 2, HIDDEN), jnp.float32),
    mesh=plsc.VectorSubcoreMesh(
        core_axis_name="c", subcore_axis_name="s", num_cores=NC, num_subcores=NS
    ),
    scratch_types=[
        pltpu.VMEM((NG, G), jnp.int32),
        pltpu.VMEM((NB, G, 2, HIDDEN), jnp.float32),
        pltpu.VMEM_SHARED((16 * REP, 2, HIDDEN), jnp.float32),
        pltpu.SemaphoreType.DMA((NB,)),
        pltpu.SemaphoreType.DMA((NB,)),
        pltpu.SemaphoreType.DMA,
    ],
)(_gather_body)


def kernel(input_rotation, table, gamma, beta):
    table_rep = _pair_table(table, gamma, beta)
    ir = input_rotation.astype(jnp.int32)
    idx2 = ir[:, 0::2] * 4 + ir[:, 1::2]
    idx = idx2.reshape(NW, NG, G)
    out = _gather(idx, table_rep)
    return out.reshape(4096, 200, HIDDEN)


# speedup vs baseline: 1.0550x; 1.0550x over previous
import functools

import jax
import jax.numpy as jnp
from jax import lax
from jax.experimental import pallas as pl
from jax.experimental.pallas import tpu as pltpu
from jax.experimental.pallas import tpu_sc as plsc

HIDDEN = 128
EPS = 1e-12

NC, NS = 2, 16
NW = NC * NS
B_TOTAL = 4096 * 200
B_PER_W = B_TOTAL // NW
G = 128
NG = B_PER_W // G
REP = 32
NB = 4
NR = NG // NB


def _ln_body(t_ref, g_ref, b_ref, o_ref):
    t = t_ref[...]
    mean = jnp.mean(t, axis=-1, keepdims=True)
    c = t - mean
    var = jnp.mean(c * c, axis=-1, keepdims=True)
    n = c * lax.rsqrt(var + EPS) * g_ref[...] + b_ref[...]
    o_ref[...] = jnp.broadcast_to(n[None], (REP, 4, HIDDEN))


def _normed_table_rep(table, gamma, beta):
    out = pl.pallas_call(
        _ln_body,
        out_shape=jax.ShapeDtypeStruct((REP, 4, HIDDEN), jnp.float32),
    )(table, gamma.reshape(1, HIDDEN), beta.reshape(1, HIDDEN))
    return out.reshape(4 * REP, HIDDEN)


def _gather_body(idx_hbm, table_hbm, out_hbm, idx_v, rows_v, spm_table,
                 gsem, ssem, isem):
    sid = lax.axis_index("s")
    wid = sid * NC + lax.axis_index("c")
    base = wid * B_PER_W
    pltpu.sync_copy(
        table_hbm.at[pl.ds(sid * (4 * REP // NS), 4 * REP // NS)],
        spm_table.at[pl.ds(sid * (4 * REP // NS), 4 * REP // NS)])

    pltpu.sync_copy(idx_hbm.at[wid, pl.ds(0, 8)], idx_v.at[pl.ds(0, 8)])
    pltpu.async_copy(
        idx_hbm.at[wid, pl.ds(8, NG - 8)], idx_v.at[pl.ds(8, NG - 8)], isem)
    plsc.subcore_barrier()

    lanes = lax.iota(jnp.int32, 16)

    def fix_row(t):
        for c in range(8):
            sl = pl.ds(c * 16, 16)
            idx_v[t, sl] = idx_v[t, sl] + (lanes * 4 + (c % 2) * 64)

    def gstart(b, j):
        pltpu.async_copy(spm_table.at[idx_v.at[j]], rows_v.at[b], gsem.at[b])

    def gwait(b, j):
        pltpu.make_async_copy(
            spm_table.at[idx_v.at[j]], rows_v.at[b], gsem.at[b]).wait()

    def sstart(b, j):
        pltpu.async_copy(
            rows_v.at[b], out_hbm.at[pl.ds(base + j * G, G)], ssem.at[b])

    def swait(b, j):
        pltpu.make_async_copy(
            rows_v.at[b], out_hbm.at[pl.ds(base + j * G, G)], ssem.at[b]).wait()

    for b in range(NB):
        fix_row(b)
        gstart(b, b)

    pltpu.make_async_copy(
        idx_hbm.at[wid, pl.ds(8, NG - 8)], idx_v.at[pl.ds(8, NG - 8)],
        isem).wait()

    def round_(r, _):
        j0 = r * NB
        for b in range(NB):
            j = j0 + b
            fix_row(j + NB)
            gwait(b, j)
            sstart(b, j)
            swait(b, j)
            gstart(b, j + NB)
        return 0

    lax.fori_loop(0, NR - 1, round_, 0)

    j0 = (NR - 1) * NB
    for b in range(NB):
        gwait(b, j0 + b)
        sstart(b, j0 + b)
        swait(b, j0 + b)


_gather = functools.partial(
    pl.kernel,
    out_type=jax.ShapeDtypeStruct((B_TOTAL, HIDDEN), jnp.float32),
    mesh=plsc.VectorSubcoreMesh(
        core_axis_name="c", subcore_axis_name="s", num_cores=NC, num_subcores=NS
    ),
    scratch_types=[
        pltpu.VMEM((NG, G), jnp.int32),
        pltpu.VMEM((NB, G, HIDDEN), jnp.float32),
        pltpu.VMEM_SHARED((4 * REP, HIDDEN), jnp.float32),
        pltpu.SemaphoreType.DMA((NB,)),
        pltpu.SemaphoreType.DMA((NB,)),
        pltpu.SemaphoreType.DMA,
    ],
)(_gather_body)


def kernel(input_rotation, table, gamma, beta):
    table_rep = _normed_table_rep(table, gamma, beta)
    idx = input_rotation.reshape(NW, NG, G).astype(jnp.int32)
    out = _gather(idx, table_rep)
    return out.reshape(4096, 200, HIDDEN)


# speedup vs baseline: 1.0552x; 1.0002x over previous
import functools

import jax
import jax.numpy as jnp
from jax import lax
from jax.experimental import pallas as pl
from jax.experimental.pallas import tpu as pltpu
from jax.experimental.pallas import tpu_sc as plsc

HIDDEN = 128
EPS = 1e-12

NC, NS = 2, 16
NW = NC * NS
B_TOTAL = 4096 * 200
B_PER_W = B_TOTAL // NW
G = 128
NG = B_PER_W // G
REP = 64
NB = 4
NR = NG // NB


def _ln_body(t_ref, g_ref, b_ref, o_ref):
    t = t_ref[...]
    mean = jnp.mean(t, axis=-1, keepdims=True)
    c = t - mean
    var = jnp.mean(c * c, axis=-1, keepdims=True)
    n = c * lax.rsqrt(var + EPS) * g_ref[...] + b_ref[...]
    o_ref[...] = jnp.broadcast_to(n[None], (REP, 4, HIDDEN))


def _normed_table_rep(table, gamma, beta):
    out = pl.pallas_call(
        _ln_body,
        out_shape=jax.ShapeDtypeStruct((REP, 4, HIDDEN), jnp.float32),
    )(table, gamma.reshape(1, HIDDEN), beta.reshape(1, HIDDEN))
    return out.reshape(4 * REP, HIDDEN)


def _gather_body(idx_hbm, table_hbm, out_hbm, idx_v, rows_v, spm_table,
                 gsem, ssem, isem):
    sid = lax.axis_index("s")
    wid = sid * NC + lax.axis_index("c")
    base = wid * B_PER_W
    pltpu.sync_copy(
        table_hbm.at[pl.ds(sid * (4 * REP // NS), 4 * REP // NS)],
        spm_table.at[pl.ds(sid * (4 * REP // NS), 4 * REP // NS)])

    pltpu.sync_copy(idx_hbm.at[wid, pl.ds(0, 8)], idx_v.at[pl.ds(0, 8)])
    pltpu.async_copy(
        idx_hbm.at[wid, pl.ds(8, NG - 8)], idx_v.at[pl.ds(8, NG - 8)], isem)
    plsc.subcore_barrier()

    lanes = lax.iota(jnp.int32, 16)

    def fix_row(t):
        for c in range(8):
            sl = pl.ds(c * 16, 16)
            idx_v[t, sl] = idx_v[t, sl] + (lanes * 4 + (c % 4) * 64)

    def gstart(b, j):
        pltpu.async_copy(spm_table.at[idx_v.at[j]], rows_v.at[b], gsem.at[b])

    def gwait(b, j):
        pltpu.make_async_copy(
            spm_table.at[idx_v.at[j]], rows_v.at[b], gsem.at[b]).wait()

    def sstart(b, j):
        pltpu.async_copy(
            rows_v.at[b], out_hbm.at[pl.ds(base + j * G, G)], ssem.at[b])

    def swait(b, j):
        pltpu.make_async_copy(
            rows_v.at[b], out_hbm.at[pl.ds(base + j * G, G)], ssem.at[b]).wait()

    for b in range(NB):
        fix_row(b)
        gstart(b, b)

    pltpu.make_async_copy(
        idx_hbm.at[wid, pl.ds(8, NG - 8)], idx_v.at[pl.ds(8, NG - 8)],
        isem).wait()

    def round_(r, _):
        j0 = r * NB
        for b in range(NB):
            j = j0 + b
            fix_row(j + NB)
            gwait(b, j)
            sstart(b, j)
            swait(b, j)
            gstart(b, j + NB)
        return 0

    lax.fori_loop(0, NR - 1, round_, 0)

    j0 = (NR - 1) * NB
    for b in range(NB):
        gwait(b, j0 + b)
        sstart(b, j0 + b)
        swait(b, j0 + b)


_gather = functools.partial(
    pl.kernel,
    out_type=jax.ShapeDtypeStruct((B_TOTAL, HIDDEN), jnp.float32),
    mesh=plsc.VectorSubcoreMesh(
        core_axis_name="c", subcore_axis_name="s", num_cores=NC, num_subcores=NS
    ),
    scratch_types=[
        pltpu.VMEM((NG, G), jnp.int32),
        pltpu.VMEM((NB, G, HIDDEN), jnp.float32),
        pltpu.VMEM_SHARED((4 * REP, HIDDEN), jnp.float32),
        pltpu.SemaphoreType.DMA((NB,)),
        pltpu.SemaphoreType.DMA((NB,)),
        pltpu.SemaphoreType.DMA,
    ],
)(_gather_body)


def kernel(input_rotation, table, gamma, beta):
    table_rep = _normed_table_rep(table, gamma, beta)
    idx = input_rotation.reshape(NW, NG, G).astype(jnp.int32)
    out = _gather(idx, table_rep)
    return out.reshape(4096, 200, HIDDEN)
